# Initial kernel scaffold; baseline (speedup 1.0000x reference)
import jax
import jax.numpy as jnp
from jax.experimental import pallas as pl


def kernel(x, edge_index, W1, b1, g1, be1, W2, b2, g2, be2, Wo, bo):
    raise NotImplementedError("write your pallas kernel here")


# speedup vs baseline: 8.5311x; 8.5311x over previous
import functools

import jax
import jax.numpy as jnp
from jax import lax
from jax.experimental import pallas as pl
from jax.experimental.pallas import tpu as pltpu
from jax.experimental.pallas import tpu_sc as plsc

N = 10000
E = 320000
D = 128
H = 256
C = 1000

NC = 2
NS = 16
NP = 10240
RPT = NP // NS
HH = H // 2

RB = 1000
NRB = N // RB

K = 80

@functools.cache
def _sc_mesh():
    return plsc.VectorSubcoreMesh(
        core_axis_name="c", subcore_axis_name="s", num_cores=NC, num_subcores=NS
    )


def _deg_body(dst_hbm, ones_hbm, zcol_hbm, out_hbm, idx_v, ones_v, acc, sem):
    c = lax.axis_index("c")
    s = lax.axis_index("s")
    ew = E // (NC * NS)
    ebase = (c * NS + s) * ew

    pltpu.sync_copy(zcol_hbm, acc.at[pl.ds(s * RPT, RPT)])
    pltpu.sync_copy(ones_hbm, ones_v)
    plsc.subcore_barrier()

    def body(i, carry):
        b = pl.multiple_of(ebase + i * K, K)
        pltpu.sync_copy(dst_hbm.at[pl.ds(b, K)], idx_v)
        pltpu.sync_copy(ones_v, acc.at[idx_v], add=True)
        return carry

    lax.fori_loop(0, ew // K, body, 0)
    plsc.subcore_barrier()
    pltpu.sync_copy(
        acc.at[pl.ds(s * RPT, RPT)],
        out_hbm.at[pl.ds(c * NP + s * RPT, RPT)],
    )


def _deg_call(*args):
    return pl.kernel(
        _deg_body,
        out_type=jax.ShapeDtypeStruct((NC * NP,), jnp.float32),
        mesh=_sc_mesh(),
        scratch_types=[
            pltpu.VMEM((K,), jnp.int32),
            pltpu.VMEM((K,), jnp.float32),
            pltpu.VMEM_SHARED((NP,), jnp.float32),
            pltpu.SemaphoreType.DMA,
        ],
    )(*args)


def _agg_body(hs_hbm, src_hbm, dst_hbm, zrows_hbm, out_hbm,
              sidx, didx, rows, acc, sem):
    c = lax.axis_index("c")
    s = lax.axis_index("s")
    et = E // NS
    ebase = s * et
    off = c * NP

    pltpu.sync_copy(zrows_hbm, acc.at[pl.ds(s * RPT, RPT)])
    plsc.subcore_barrier()

    def body(i, carry):
        b = pl.multiple_of(ebase + i * K, K)
        pltpu.sync_copy(src_hbm.at[pl.ds(b, K)], sidx)
        pltpu.sync_copy(dst_hbm.at[pl.ds(b, K)], didx)
        for j in range(K // 16):
            sidx[pl.ds(j * 16, 16)] = sidx[pl.ds(j * 16, 16)] + off
        pltpu.async_copy(hs_hbm.at[sidx], rows, sem).wait()
        pltpu.sync_copy(rows, acc.at[didx], add=True)
        return carry

    lax.fori_loop(0, et // K, body, 0)
    plsc.subcore_barrier()
    pltpu.sync_copy(
        acc.at[pl.ds(s * RPT, RPT)],
        out_hbm.at[pl.ds(c * NP + s * RPT, RPT)],
    )


def _agg_call(*args):
    return pl.kernel(
        _agg_body,
        out_type=jax.ShapeDtypeStruct((NC * NP, HH), jnp.float32),
        mesh=_sc_mesh(),
        scratch_types=[
            pltpu.VMEM((K,), jnp.int32),
            pltpu.VMEM((K,), jnp.int32),
            pltpu.VMEM((K, HH), jnp.float32),
            pltpu.VMEM_SHARED((NP, HH), jnp.float32),
            pltpu.SemaphoreType.DMA,
        ],
    )(*args)


def _dis_of(degcol_blk):
    return lax.rsqrt(degcol_blk + 1.0)


def _mm1_body(x_ref, w_ref, deg_ref, hs_ref):
    x = x_ref[...]
    nrm = jnp.sqrt(jnp.sum(x * x, axis=1, keepdims=True))
    xn = x / jnp.maximum(nrm, 1e-12)
    h = jnp.dot(xn, w_ref[...], preferred_element_type=jnp.float32)
    hs = h * _dis_of(deg_ref[...])
    hs_ref[0] = hs[:, :HH]
    hs_ref[1] = hs[:, HH:]


def _mm1_call(x, W1, degcol):
    return pl.pallas_call(
        _mm1_body,
        grid=(NRB,),
        in_specs=[
            pl.BlockSpec((RB, D), lambda i: (i, 0)),
            pl.BlockSpec((D, H), lambda i: (0, 0)),
            pl.BlockSpec((RB, 1), lambda i: (i, 0)),
        ],
        out_specs=pl.BlockSpec((2, RB, HH), lambda i: (0, i, 0)),
        out_shape=jax.ShapeDtypeStruct((2, NP, HH), jnp.float32),
    )(x, W1, degcol)


def _stats_body(t_ref, hs_ref, deg_ref, b_ref, z_ref, st_ref, acc):
    i = pl.program_id(0)
    t = jnp.concatenate([t_ref[0], t_ref[1]], axis=1)
    hs = jnp.concatenate([hs_ref[0], hs_ref[1]], axis=1)
    z = _dis_of(deg_ref[...]) * (t + hs) + b_ref[...]
    z_ref[...] = z

    @pl.when(i == 0)
    def _():
        acc[...] = jnp.zeros_like(acc)

    acc[0:1] += jnp.sum(z, axis=0, keepdims=True)
    acc[1:2] += jnp.sum(z * z, axis=0, keepdims=True)

    @pl.when(i == NRB - 1)
    def _():
        st_ref[...] = acc[...]


def _stats_call(tT, hsT, degcol, brow):
    return pl.pallas_call(
        _stats_body,
        grid=(NRB,),
        in_specs=[
            pl.BlockSpec((2, RB, HH), lambda i: (0, i, 0)),
            pl.BlockSpec((2, RB, HH), lambda i: (0, i, 0)),
            pl.BlockSpec((RB, 1), lambda i: (i, 0)),
            pl.BlockSpec((1, H), lambda i: (0, 0)),
        ],
        out_specs=[
            pl.BlockSpec((RB, H), lambda i: (i, 0)),
            pl.BlockSpec((2, H), lambda i: (0, 0)),
        ],
        out_shape=[
            jax.ShapeDtypeStruct((N, H), jnp.float32),
            jax.ShapeDtypeStruct((2, H), jnp.float32),
        ],
        scratch_shapes=[pltpu.VMEM((2, H), jnp.float32)],
    )(tT, hsT, degcol, brow)


def _bn_relu(z, st_ref, g_ref, be_ref):
    m = st_ref[0:1] * (1.0 / N)
    var = st_ref[1:2] * (1.0 / N) - m * m
    rs = lax.rsqrt(var + 1e-5)
    return jnp.maximum((z - m) * rs * g_ref[...] + be_ref[...], 0.0)


def _mm2_body(z_ref, st_ref, g_ref, be_ref, w_ref, deg_ref, hs_ref):
    a = _bn_relu(z_ref[...], st_ref, g_ref, be_ref)
    h = jnp.dot(a, w_ref[...], preferred_element_type=jnp.float32)
    hs = h * _dis_of(deg_ref[...])
    hs_ref[0] = hs[:, :HH]
    hs_ref[1] = hs[:, HH:]


def _mm2_call(z, st, g, be, W2, degcol):
    return pl.pallas_call(
        _mm2_body,
        grid=(NRB,),
        in_specs=[
            pl.BlockSpec((RB, H), lambda i: (i, 0)),
            pl.BlockSpec((2, H), lambda i: (0, 0)),
            pl.BlockSpec((1, H), lambda i: (0, 0)),
            pl.BlockSpec((1, H), lambda i: (0, 0)),
            pl.BlockSpec((H, H), lambda i: (0, 0)),
            pl.BlockSpec((RB, 1), lambda i: (i, 0)),
        ],
        out_specs=pl.BlockSpec((2, RB, HH), lambda i: (0, i, 0)),
        out_shape=jax.ShapeDtypeStruct((2, NP, HH), jnp.float32),
    )(z, st, g, be, W2, degcol)


def _out_body(z_ref, st_ref, g_ref, be_ref, w_ref, bo_ref, o_ref):
    a = _bn_relu(z_ref[...], st_ref, g_ref, be_ref)
    o = jnp.dot(a, w_ref[...], preferred_element_type=jnp.float32)
    o = o + bo_ref[...]
    m = jnp.max(o, axis=1, keepdims=True)
    e = jnp.exp(o - m)
    o_ref[...] = e / jnp.sum(e, axis=1, keepdims=True)


def _out_call(z, st, g, be, WoT, borow):
    return pl.pallas_call(
        _out_body,
        grid=(NRB,),
        in_specs=[
            pl.BlockSpec((RB, H), lambda i: (i, 0)),
            pl.BlockSpec((2, H), lambda i: (0, 0)),
            pl.BlockSpec((1, H), lambda i: (0, 0)),
            pl.BlockSpec((1, H), lambda i: (0, 0)),
            pl.BlockSpec((H, C), lambda i: (0, 0)),
            pl.BlockSpec((1, C), lambda i: (0, 0)),
        ],
        out_specs=pl.BlockSpec((RB, C), lambda i: (i, 0)),
        out_shape=jax.ShapeDtypeStruct((N, C), jnp.float32),
    )(z, st, g, be, WoT, borow)


def kernel(x, edge_index, W1, b1, g1, be1, W2, b2, g2, be2, Wo, bo):
    src = edge_index[0]
    dst = edge_index[1]

    ones_k = jnp.ones((K,), jnp.float32)
    zcol = jnp.zeros((RPT,), jnp.float32)
    zrows = jnp.zeros((RPT, HH), jnp.float32)

    degp = _deg_call(dst, ones_k, zcol)
    degcol = (degp[:NP] + degp[NP:])[:N, None]

    hs1T = _mm1_call(x, W1, degcol)
    t1 = _agg_call(hs1T.reshape(NC * NP, HH), src, dst, zrows)
    z1, st1 = _stats_call(t1.reshape(NC, NP, HH), hs1T, degcol,
                          b1.reshape(1, H))

    hs2T = _mm2_call(z1, st1, g1.reshape(1, H), be1.reshape(1, H), W2, degcol)
    t2 = _agg_call(hs2T.reshape(NC * NP, HH), src, dst, zrows)
    z2, st2 = _stats_call(t2.reshape(NC, NP, HH), hs2T, degcol,
                          b2.reshape(1, H))

    return _out_call(z2, st2, g2.reshape(1, H), be2.reshape(1, H),
                     Wo.T, bo.reshape(1, C))


# speedup vs baseline: 15.6799x; 1.8380x over previous
import functools

import jax
import jax.numpy as jnp
from jax import lax
from jax.experimental import pallas as pl
from jax.experimental.pallas import tpu as pltpu
from jax.experimental.pallas import tpu_sc as plsc

N = 10000
E = 320000
D = 128
H = 256
C = 1000

NC = 2
NS = 16
NP = 10240
RPT = NP // NS
HH = H // 2

RB = 1000
NRB = N // RB

K = 80

@functools.cache
def _sc_mesh():
    return plsc.VectorSubcoreMesh(
        core_axis_name="c", subcore_axis_name="s", num_cores=NC, num_subcores=NS
    )


EW = E // (NC * NS)
NCHD = EW // K
DGRP = 5
ET = E // NS
NCH = ET // K
SEG = 50
NSEG = NCH // SEG


def _deg_body(dst_hbm, ones_hbm, zcol_hbm, out_hbm, idxb, ones_v, acc, sem):
    c = lax.axis_index("c")
    s = lax.axis_index("s")
    w = c * NS + s

    pltpu.sync_copy(zcol_hbm, acc.at[pl.ds(s * RPT, RPT)])
    pltpu.sync_copy(dst_hbm.at[w], idxb)
    pltpu.sync_copy(ones_hbm, ones_v)
    plsc.subcore_barrier()

    def body(g, carry):
        for b in range(DGRP):
            ch = g * DGRP + b
            pltpu.async_copy(ones_v, acc.at[idxb.at[ch]], sem, add=True)
        for b in range(DGRP):
            ch = g * DGRP + b
            pltpu.make_async_copy(ones_v, acc.at[idxb.at[ch]], sem).wait()
        return carry

    lax.fori_loop(0, NCHD // DGRP, body, 0)
    plsc.subcore_barrier()
    pltpu.sync_copy(
        acc.at[pl.ds(s * RPT, RPT)],
        out_hbm.at[pl.ds(c * NP + s * RPT, RPT)],
    )


def _deg_call(*args):
    return pl.kernel(
        _deg_body,
        out_type=jax.ShapeDtypeStruct((NC * NP,), jnp.float32),
        mesh=_sc_mesh(),
        scratch_types=[
            pltpu.VMEM((NCHD, K), jnp.int32),
            pltpu.VMEM((K,), jnp.float32),
            pltpu.VMEM_SHARED((NP,), jnp.float32),
            pltpu.SemaphoreType.DMA,
        ],
    )(*args)


def _agg_body(hs_hbm, src_hbm, dst_hbm, zrows_hbm, out_hbm,
              sidxb, didxb, rows, acc, gsem, ssem):
    c = lax.axis_index("c")
    s = lax.axis_index("s")
    off = c * NP

    pltpu.sync_copy(zrows_hbm, acc.at[pl.ds(s * RPT, RPT)])
    plsc.subcore_barrier()

    def seg_body(sg, carry):
        base = s * NSEG + sg
        pltpu.sync_copy(src_hbm.at[base], sidxb)
        pltpu.sync_copy(dst_hbm.at[base], didxb)

        def offset_body(i, c2):
            for j in range(K // 16):
                sl = pl.ds(j * 16, 16)
                sidxb[i, sl] = sidxb[i, sl] + off
            return c2

        lax.fori_loop(0, SEG, offset_body, 0)

        pltpu.async_copy(hs_hbm.at[sidxb.at[0]], rows[0], gsem[0])

        def group(g, c2):
            for b in range(2):
                ch = 2 * g + b
                nb = 1 - b
                pltpu.make_async_copy(
                    hs_hbm.at[sidxb.at[ch]], rows[b], gsem[b]
                ).wait()

                @pl.when(ch >= 1)
                def _():
                    pltpu.make_async_copy(
                        rows[nb], acc.at[didxb.at[ch - 1]], ssem[nb]
                    ).wait()

                pltpu.async_copy(rows[b], acc.at[didxb.at[ch]], ssem[b],
                                 add=True)

                @pl.when(ch + 1 < SEG)
                def _():
                    pltpu.async_copy(hs_hbm.at[sidxb.at[ch + 1]], rows[nb],
                                     gsem[nb])
            return c2

        lax.fori_loop(0, SEG // 2, group, 0)
        pltpu.make_async_copy(
            rows[1], acc.at[didxb.at[SEG - 1]], ssem[1]
        ).wait()
        return carry

    lax.fori_loop(0, NSEG, seg_body, 0)
    plsc.subcore_barrier()
    pltpu.sync_copy(
        acc.at[pl.ds(s * RPT, RPT)],
        out_hbm.at[pl.ds(c * NP + s * RPT, RPT)],
    )


def _agg_call(*args):
    return pl.kernel(
        _agg_body,
        out_type=jax.ShapeDtypeStruct((NC * NP, HH), jnp.float32),
        mesh=_sc_mesh(),
        scratch_types=[
            pltpu.VMEM((SEG, K), jnp.int32),
            pltpu.VMEM((SEG, K), jnp.int32),
            [pltpu.VMEM((K, HH), jnp.float32) for _ in range(2)],
            pltpu.VMEM_SHARED((NP, HH), jnp.float32),
            [pltpu.SemaphoreType.DMA for _ in range(2)],
            [pltpu.SemaphoreType.DMA for _ in range(2)],
        ],
    )(*args)


def _dis_of(degcol_blk):
    return lax.rsqrt(degcol_blk + 1.0)


def _mm1_body(x_ref, w_ref, deg_ref, hs_ref):
    x = x_ref[...]
    nrm = jnp.sqrt(jnp.sum(x * x, axis=1, keepdims=True))
    xn = x / jnp.maximum(nrm, 1e-12)
    h = jnp.dot(xn, w_ref[...], preferred_element_type=jnp.float32)
    hs = h * _dis_of(deg_ref[...])
    hs_ref[0] = hs[:, :HH]
    hs_ref[1] = hs[:, HH:]


def _mm1_call(x, W1, degcol):
    return pl.pallas_call(
        _mm1_body,
        grid=(NRB,),
        in_specs=[
            pl.BlockSpec((RB, D), lambda i: (i, 0)),
            pl.BlockSpec((D, H), lambda i: (0, 0)),
            pl.BlockSpec((RB, 1), lambda i: (i, 0)),
        ],
        out_specs=pl.BlockSpec((2, RB, HH), lambda i: (0, i, 0)),
        out_shape=jax.ShapeDtypeStruct((2, NP, HH), jnp.float32),
    )(x, W1, degcol)


def _stats_body(t_ref, hs_ref, deg_ref, b_ref, z_ref, st_ref, acc):
    i = pl.program_id(0)
    t = jnp.concatenate([t_ref[0], t_ref[1]], axis=1)
    hs = jnp.concatenate([hs_ref[0], hs_ref[1]], axis=1)
    z = _dis_of(deg_ref[...]) * (t + hs) + b_ref[...]
    z_ref[...] = z

    @pl.when(i == 0)
    def _():
        acc[...] = jnp.zeros_like(acc)

    acc[0:1] += jnp.sum(z, axis=0, keepdims=True)
    acc[1:2] += jnp.sum(z * z, axis=0, keepdims=True)

    @pl.when(i == NRB - 1)
    def _():
        st_ref[...] = acc[...]


def _stats_call(tT, hsT, degcol, brow):
    return pl.pallas_call(
        _stats_body,
        grid=(NRB,),
        in_specs=[
            pl.BlockSpec((2, RB, HH), lambda i: (0, i, 0)),
            pl.BlockSpec((2, RB, HH), lambda i: (0, i, 0)),
            pl.BlockSpec((RB, 1), lambda i: (i, 0)),
            pl.BlockSpec((1, H), lambda i: (0, 0)),
        ],
        out_specs=[
            pl.BlockSpec((RB, H), lambda i: (i, 0)),
            pl.BlockSpec((2, H), lambda i: (0, 0)),
        ],
        out_shape=[
            jax.ShapeDtypeStruct((N, H), jnp.float32),
            jax.ShapeDtypeStruct((2, H), jnp.float32),
        ],
        scratch_shapes=[pltpu.VMEM((2, H), jnp.float32)],
    )(tT, hsT, degcol, brow)


def _bn_relu(z, st_ref, g_ref, be_ref):
    m = st_ref[0:1] * (1.0 / N)
    var = st_ref[1:2] * (1.0 / N) - m * m
    rs = lax.rsqrt(var + 1e-5)
    return jnp.maximum((z - m) * rs * g_ref[...] + be_ref[...], 0.0)


def _mm2_body(z_ref, st_ref, g_ref, be_ref, w_ref, deg_ref, hs_ref):
    a = _bn_relu(z_ref[...], st_ref, g_ref, be_ref)
    h = jnp.dot(a, w_ref[...], preferred_element_type=jnp.float32)
    hs = h * _dis_of(deg_ref[...])
    hs_ref[0] = hs[:, :HH]
    hs_ref[1] = hs[:, HH:]


def _mm2_call(z, st, g, be, W2, degcol):
    return pl.pallas_call(
        _mm2_body,
        grid=(NRB,),
        in_specs=[
            pl.BlockSpec((RB, H), lambda i: (i, 0)),
            pl.BlockSpec((2, H), lambda i: (0, 0)),
            pl.BlockSpec((1, H), lambda i: (0, 0)),
            pl.BlockSpec((1, H), lambda i: (0, 0)),
            pl.BlockSpec((H, H), lambda i: (0, 0)),
            pl.BlockSpec((RB, 1), lambda i: (i, 0)),
        ],
        out_specs=pl.BlockSpec((2, RB, HH), lambda i: (0, i, 0)),
        out_shape=jax.ShapeDtypeStruct((2, NP, HH), jnp.float32),
    )(z, st, g, be, W2, degcol)


def _out_body(z_ref, st_ref, g_ref, be_ref, w_ref, bo_ref, o_ref):
    a = _bn_relu(z_ref[...], st_ref, g_ref, be_ref)
    o = jnp.dot(a, w_ref[...], preferred_element_type=jnp.float32)
    o = o + bo_ref[...]
    m = jnp.max(o, axis=1, keepdims=True)
    e = jnp.exp(o - m)
    o_ref[...] = e / jnp.sum(e, axis=1, keepdims=True)


def _out_call(z, st, g, be, WoT, borow):
    return pl.pallas_call(
        _out_body,
        grid=(NRB,),
        in_specs=[
            pl.BlockSpec((RB, H), lambda i: (i, 0)),
            pl.BlockSpec((2, H), lambda i: (0, 0)),
            pl.BlockSpec((1, H), lambda i: (0, 0)),
            pl.BlockSpec((1, H), lambda i: (0, 0)),
            pl.BlockSpec((H, C), lambda i: (0, 0)),
            pl.BlockSpec((1, C), lambda i: (0, 0)),
        ],
        out_specs=pl.BlockSpec((RB, C), lambda i: (i, 0)),
        out_shape=jax.ShapeDtypeStruct((N, C), jnp.float32),
    )(z, st, g, be, WoT, borow)


def kernel(x, edge_index, W1, b1, g1, be1, W2, b2, g2, be2, Wo, bo):
    src = edge_index[0]
    dst = edge_index[1]

    ones_k = jnp.ones((K,), jnp.float32)
    zcol = jnp.zeros((RPT,), jnp.float32)
    zrows = jnp.zeros((RPT, HH), jnp.float32)

    dst_d = dst.reshape(NC * NS, NCHD, K)
    src_t = src.reshape(NS * NSEG, SEG, K)
    dst_t = dst.reshape(NS * NSEG, SEG, K)

    degp = _deg_call(dst_d, ones_k, zcol)
    degcol = (degp[:NP] + degp[NP:])[:N, None]

    hs1T = _mm1_call(x, W1, degcol)
    t1 = _agg_call(hs1T.reshape(NC * NP, HH), src_t, dst_t, zrows)
    z1, st1 = _stats_call(t1.reshape(NC, NP, HH), hs1T, degcol,
                          b1.reshape(1, H))

    hs2T = _mm2_call(z1, st1, g1.reshape(1, H), be1.reshape(1, H), W2, degcol)
    t2 = _agg_call(hs2T.reshape(NC * NP, HH), src_t, dst_t, zrows)
    z2, st2 = _stats_call(t2.reshape(NC, NP, HH), hs2T, degcol,
                          b2.reshape(1, H))

    return _out_call(z2, st2, g2.reshape(1, H), be2.reshape(1, H),
                     Wo.T, bo.reshape(1, C))


# speedup vs baseline: 19.1199x; 1.2194x over previous
import functools

import jax
import jax.numpy as jnp
from jax import lax
from jax.experimental import pallas as pl
from jax.experimental.pallas import tpu as pltpu
from jax.experimental.pallas import tpu_sc as plsc

N = 10000
E = 320000
D = 128
H = 256
C = 1000

NC = 2
NS = 16
NP = 10240
RPT = NP // NS
HH = H // 2

RB = 1000
NRB = N // RB

K = 80

@functools.cache
def _sc_mesh():
    return plsc.VectorSubcoreMesh(
        core_axis_name="c", subcore_axis_name="s", num_cores=NC, num_subcores=NS
    )


EW = E // (NC * NS)
NCHD = EW // K
DGRP = 5
ET = E // NS
NCH = ET // K
SEG = 10
NSEG = NCH // SEG


def _deg_body(dst_hbm, ones_hbm, zcol_hbm, out_hbm, idxb, ones_v, acc, sem):
    c = lax.axis_index("c")
    s = lax.axis_index("s")
    w = c * NS + s

    pltpu.sync_copy(zcol_hbm, acc.at[pl.ds(s * RPT, RPT)])
    pltpu.sync_copy(dst_hbm.at[w], idxb)
    pltpu.sync_copy(ones_hbm, ones_v)
    plsc.subcore_barrier()

    def body(g, carry):
        for b in range(DGRP):
            ch = g * DGRP + b
            pltpu.async_copy(ones_v, acc.at[idxb.at[ch]], sem, add=True)
        for b in range(DGRP):
            ch = g * DGRP + b
            pltpu.make_async_copy(ones_v, acc.at[idxb.at[ch]], sem).wait()
        return carry

    lax.fori_loop(0, NCHD // DGRP, body, 0)
    plsc.subcore_barrier()
    pltpu.sync_copy(
        acc.at[pl.ds(s * RPT, RPT)],
        out_hbm.at[pl.ds(c * NP + s * RPT, RPT)],
    )


def _deg_call(*args):
    return pl.kernel(
        _deg_body,
        out_type=jax.ShapeDtypeStruct((NC * NP,), jnp.float32),
        mesh=_sc_mesh(),
        scratch_types=[
            pltpu.VMEM((NCHD, K), jnp.int32),
            pltpu.VMEM((K,), jnp.float32),
            pltpu.VMEM_SHARED((NP,), jnp.float32),
            pltpu.SemaphoreType.DMA,
        ],
    )(*args)


def _agg_body(hs_hbm, src_hbm, dst_hbm, zrows_hbm, out_hbm,
              sidxb, didxb, rows, acc, gsem, ssem):
    c = lax.axis_index("c")
    s = lax.axis_index("s")

    pltpu.sync_copy(zrows_hbm, acc.at[pl.ds(s * RPT, RPT)])
    plsc.subcore_barrier()

    def seg_body(sg, carry):
        pltpu.sync_copy(src_hbm.at[(c * NS + s) * NSEG + sg], sidxb)
        pltpu.sync_copy(dst_hbm.at[s * NSEG + sg], didxb)

        pltpu.async_copy(hs_hbm.at[sidxb.at[0]], rows[0], gsem[0])
        pltpu.async_copy(hs_hbm.at[sidxb.at[1]], rows[1], gsem[1])
        for ch in range(SEG):
            b = ch % 3
            pltpu.make_async_copy(
                hs_hbm.at[sidxb.at[ch]], rows[b], gsem[b]
            ).wait()
            if ch >= 1:
                pb = (ch - 1) % 3
                pltpu.make_async_copy(
                    rows[pb], acc.at[didxb.at[ch - 1]], ssem[pb]
                ).wait()
            if ch + 2 < SEG:
                nb = (ch + 2) % 3
                pltpu.async_copy(hs_hbm.at[sidxb.at[ch + 2]], rows[nb],
                                 gsem[nb])
            pltpu.async_copy(rows[b], acc.at[didxb.at[ch]], ssem[b],
                             add=True)
        pltpu.make_async_copy(
            rows[(SEG - 1) % 3], acc.at[didxb.at[SEG - 1]],
            ssem[(SEG - 1) % 3]
        ).wait()
        return carry

    lax.fori_loop(0, NSEG, seg_body, 0)
    plsc.subcore_barrier()
    pltpu.sync_copy(
        acc.at[pl.ds(s * RPT, RPT)],
        out_hbm.at[pl.ds(c * NP + s * RPT, RPT)],
    )


def _agg_call(*args):
    return pl.kernel(
        _agg_body,
        out_type=jax.ShapeDtypeStruct((NC * NP, HH), jnp.float32),
        mesh=_sc_mesh(),
        scratch_types=[
            pltpu.VMEM((SEG, K), jnp.int32),
            pltpu.VMEM((SEG, K), jnp.int32),
            [pltpu.VMEM((K, HH), jnp.float32) for _ in range(3)],
            pltpu.VMEM_SHARED((NP, HH), jnp.float32),
            [pltpu.SemaphoreType.DMA for _ in range(3)],
            [pltpu.SemaphoreType.DMA for _ in range(3)],
        ],
    )(*args)


def _dis_of(degcol_blk):
    return lax.rsqrt(degcol_blk + 1.0)


def _mm1_body(x_ref, w_ref, deg_ref, hs_ref):
    x = x_ref[...]
    nrm = jnp.sqrt(jnp.sum(x * x, axis=1, keepdims=True))
    xn = x / jnp.maximum(nrm, 1e-12)
    h = jnp.dot(xn, w_ref[...], preferred_element_type=jnp.float32)
    hs = h * _dis_of(deg_ref[...])
    hs_ref[0] = hs[:, :HH]
    hs_ref[1] = hs[:, HH:]


def _mm1_call(x, W1, degcol):
    return pl.pallas_call(
        _mm1_body,
        grid=(NRB,),
        in_specs=[
            pl.BlockSpec((RB, D), lambda i: (i, 0)),
            pl.BlockSpec((D, H), lambda i: (0, 0)),
            pl.BlockSpec((RB, 1), lambda i: (i, 0)),
        ],
        out_specs=pl.BlockSpec((2, RB, HH), lambda i: (0, i, 0)),
        out_shape=jax.ShapeDtypeStruct((2, NP, HH), jnp.float32),
    )(x, W1, degcol)


def _stats_body(t_ref, hs_ref, deg_ref, b_ref, z_ref, st_ref, acc):
    i = pl.program_id(0)
    t = jnp.concatenate([t_ref[0], t_ref[1]], axis=1)
    hs = jnp.concatenate([hs_ref[0], hs_ref[1]], axis=1)
    z = _dis_of(deg_ref[...]) * (t + hs) + b_ref[...]
    z_ref[...] = z

    @pl.when(i == 0)
    def _():
        acc[...] = jnp.zeros_like(acc)

    acc[0:1] += jnp.sum(z, axis=0, keepdims=True)
    acc[1:2] += jnp.sum(z * z, axis=0, keepdims=True)

    @pl.when(i == NRB - 1)
    def _():
        st_ref[...] = acc[...]


def _stats_call(tT, hsT, degcol, brow):
    return pl.pallas_call(
        _stats_body,
        grid=(NRB,),
        in_specs=[
            pl.BlockSpec((2, RB, HH), lambda i: (0, i, 0)),
            pl.BlockSpec((2, RB, HH), lambda i: (0, i, 0)),
            pl.BlockSpec((RB, 1), lambda i: (i, 0)),
            pl.BlockSpec((1, H), lambda i: (0, 0)),
        ],
        out_specs=[
            pl.BlockSpec((RB, H), lambda i: (i, 0)),
            pl.BlockSpec((2, H), lambda i: (0, 0)),
        ],
        out_shape=[
            jax.ShapeDtypeStruct((N, H), jnp.float32),
            jax.ShapeDtypeStruct((2, H), jnp.float32),
        ],
        scratch_shapes=[pltpu.VMEM((2, H), jnp.float32)],
    )(tT, hsT, degcol, brow)


def _bn_relu(z, st_ref, g_ref, be_ref):
    m = st_ref[0:1] * (1.0 / N)
    var = st_ref[1:2] * (1.0 / N) - m * m
    rs = lax.rsqrt(var + 1e-5)
    return jnp.maximum((z - m) * rs * g_ref[...] + be_ref[...], 0.0)


def _mm2_body(z_ref, st_ref, g_ref, be_ref, w_ref, deg_ref, hs_ref):
    a = _bn_relu(z_ref[...], st_ref, g_ref, be_ref)
    h = jnp.dot(a, w_ref[...], preferred_element_type=jnp.float32)
    hs = h * _dis_of(deg_ref[...])
    hs_ref[0] = hs[:, :HH]
    hs_ref[1] = hs[:, HH:]


def _mm2_call(z, st, g, be, W2, degcol):
    return pl.pallas_call(
        _mm2_body,
        grid=(NRB,),
        in_specs=[
            pl.BlockSpec((RB, H), lambda i: (i, 0)),
            pl.BlockSpec((2, H), lambda i: (0, 0)),
            pl.BlockSpec((1, H), lambda i: (0, 0)),
            pl.BlockSpec((1, H), lambda i: (0, 0)),
            pl.BlockSpec((H, H), lambda i: (0, 0)),
            pl.BlockSpec((RB, 1), lambda i: (i, 0)),
        ],
        out_specs=pl.BlockSpec((2, RB, HH), lambda i: (0, i, 0)),
        out_shape=jax.ShapeDtypeStruct((2, NP, HH), jnp.float32),
    )(z, st, g, be, W2, degcol)


def _out_body(z_ref, st_ref, g_ref, be_ref, w_ref, bo_ref, o_ref):
    a = _bn_relu(z_ref[...], st_ref, g_ref, be_ref)
    o = jnp.dot(a, w_ref[...], preferred_element_type=jnp.float32)
    o = o + bo_ref[...]
    m = jnp.max(o, axis=1, keepdims=True)
    e = jnp.exp(o - m)
    o_ref[...] = e / jnp.sum(e, axis=1, keepdims=True)


def _out_call(z, st, g, be, WoT, borow):
    return pl.pallas_call(
        _out_body,
        grid=(NRB,),
        in_specs=[
            pl.BlockSpec((RB, H), lambda i: (i, 0)),
            pl.BlockSpec((2, H), lambda i: (0, 0)),
            pl.BlockSpec((1, H), lambda i: (0, 0)),
            pl.BlockSpec((1, H), lambda i: (0, 0)),
            pl.BlockSpec((H, C), lambda i: (0, 0)),
            pl.BlockSpec((1, C), lambda i: (0, 0)),
        ],
        out_specs=pl.BlockSpec((RB, C), lambda i: (i, 0)),
        out_shape=jax.ShapeDtypeStruct((N, C), jnp.float32),
    )(z, st, g, be, WoT, borow)


def kernel(x, edge_index, W1, b1, g1, be1, W2, b2, g2, be2, Wo, bo):
    src = edge_index[0]
    dst = edge_index[1]

    ones_k = jnp.ones((K,), jnp.float32)
    zcol = jnp.zeros((RPT,), jnp.float32)
    zrows = jnp.zeros((RPT, HH), jnp.float32)

    dst_d = dst.reshape(NC * NS, NCHD, K)
    src_t = jnp.stack([src, src + NP]).reshape(NC * NS * NSEG, SEG, K)
    dst_t = dst.reshape(NS * NSEG, SEG, K)

    degp = _deg_call(dst_d, ones_k, zcol)
    degcol = (degp[:NP] + degp[NP:])[:N, None]

    hs1T = _mm1_call(x, W1, degcol)
    t1 = _agg_call(hs1T.reshape(NC * NP, HH), src_t, dst_t, zrows)
    z1, st1 = _stats_call(t1.reshape(NC, NP, HH), hs1T, degcol,
                          b1.reshape(1, H))

    hs2T = _mm2_call(z1, st1, g1.reshape(1, H), be1.reshape(1, H), W2, degcol)
    t2 = _agg_call(hs2T.reshape(NC * NP, HH), src_t, dst_t, zrows)
    z2, st2 = _stats_call(t2.reshape(NC, NP, HH), hs2T, degcol,
                          b2.reshape(1, H))

    return _out_call(z2, st2, g2.reshape(1, H), be2.reshape(1, H),
                     Wo.T, bo.reshape(1, C))


# speedup vs baseline: 19.4345x; 1.0165x over previous
import functools

import jax
import jax.numpy as jnp
from jax import lax
from jax.experimental import pallas as pl
from jax.experimental.pallas import tpu as pltpu
from jax.experimental.pallas import tpu_sc as plsc

N = 10000
E = 320000
D = 128
H = 256
C = 1000

NC = 2
NS = 16
NP = 10240
RPT = NP // NS
HH = H // 2

RB = 1000
NRB = N // RB

K = 80

@functools.cache
def _sc_mesh():
    return plsc.VectorSubcoreMesh(
        core_axis_name="c", subcore_axis_name="s", num_cores=NC, num_subcores=NS
    )


EW = E // (NC * NS)
NCHD = EW // K
DGRP = 5
ET = E // NS
NCH = ET // K
SEG = 10
NSEG = NCH // SEG


def _deg_body(dst_hbm, ones_hbm, zcol_hbm, out_hbm, idxb, ones_v, acc, sem):
    c = lax.axis_index("c")
    s = lax.axis_index("s")
    w = c * NS + s

    pltpu.sync_copy(zcol_hbm, acc.at[pl.ds(s * RPT, RPT)])
    pltpu.sync_copy(dst_hbm.at[w], idxb)
    pltpu.sync_copy(ones_hbm, ones_v)
    plsc.subcore_barrier()

    def body(g, carry):
        for b in range(DGRP):
            ch = g * DGRP + b
            pltpu.async_copy(ones_v, acc.at[idxb.at[ch]], sem, add=True)
        for b in range(DGRP):
            ch = g * DGRP + b
            pltpu.make_async_copy(ones_v, acc.at[idxb.at[ch]], sem).wait()
        return carry

    lax.fori_loop(0, NCHD // DGRP, body, 0)
    plsc.subcore_barrier()
    pltpu.sync_copy(
        acc.at[pl.ds(s * RPT, RPT)],
        out_hbm.at[pl.ds(c * NP + s * RPT, RPT)],
    )


def _deg_call(*args):
    return pl.kernel(
        _deg_body,
        out_type=jax.ShapeDtypeStruct((NC * NP,), jnp.float32),
        mesh=_sc_mesh(),
        scratch_types=[
            pltpu.VMEM((NCHD, K), jnp.int32),
            pltpu.VMEM((K,), jnp.float32),
            pltpu.VMEM_SHARED((NP,), jnp.float32),
            pltpu.SemaphoreType.DMA,
        ],
    )(*args)


def _agg_body(hs_hbm, src_hbm, dst_hbm, zrows_hbm, out_hbm,
              sidxb, didxb, rows, acc, gsem, ssem):
    c = lax.axis_index("c")
    s = lax.axis_index("s")

    pltpu.sync_copy(zrows_hbm, acc.at[pl.ds(s * RPT, RPT)])
    plsc.subcore_barrier()

    def seg_body(sg, carry):
        pltpu.sync_copy(src_hbm.at[(c * NS + s) * NSEG + sg], sidxb)
        pltpu.sync_copy(dst_hbm.at[s * NSEG + sg], didxb)

        pltpu.async_copy(hs_hbm.at[sidxb.at[0]], rows[0], gsem[0])
        pltpu.async_copy(hs_hbm.at[sidxb.at[1]], rows[1], gsem[1])
        for ch in range(SEG):
            b = ch % 3
            pltpu.make_async_copy(
                hs_hbm.at[sidxb.at[ch]], rows[b], gsem[b]
            ).wait()
            if ch >= 1:
                pb = (ch - 1) % 3
                pltpu.make_async_copy(
                    rows[pb], acc.at[didxb.at[ch - 1]], ssem[pb]
                ).wait()
            if ch + 2 < SEG:
                nb = (ch + 2) % 3
                pltpu.async_copy(hs_hbm.at[sidxb.at[ch + 2]], rows[nb],
                                 gsem[nb])
            pltpu.async_copy(rows[b], acc.at[didxb.at[ch]], ssem[b],
                             add=True)
        pltpu.make_async_copy(
            rows[(SEG - 1) % 3], acc.at[didxb.at[SEG - 1]],
            ssem[(SEG - 1) % 3]
        ).wait()
        return carry

    lax.fori_loop(0, NSEG, seg_body, 0)
    plsc.subcore_barrier()
    pltpu.sync_copy(
        acc.at[pl.ds(s * RPT, RPT)],
        out_hbm.at[pl.ds(c * NP + s * RPT, RPT)],
    )


def _agg_call(*args):
    return pl.kernel(
        _agg_body,
        out_type=jax.ShapeDtypeStruct((NC * NP, HH), jnp.float32),
        mesh=_sc_mesh(),
        scratch_types=[
            pltpu.VMEM((SEG, K), jnp.int32),
            pltpu.VMEM((SEG, K), jnp.int32),
            [pltpu.VMEM((K, HH), jnp.float32) for _ in range(3)],
            pltpu.VMEM_SHARED((NP, HH), jnp.float32),
            [pltpu.SemaphoreType.DMA for _ in range(3)],
            [pltpu.SemaphoreType.DMA for _ in range(3)],
        ],
    )(*args)


def _dis_of(degcol_blk):
    return lax.rsqrt(degcol_blk + 1.0)


def _mm1_body(x_ref, w_ref, deg_ref, hs_ref):
    x = x_ref[...]
    nrm = jnp.sqrt(jnp.sum(x * x, axis=1, keepdims=True))
    xn = x / jnp.maximum(nrm, 1e-12)
    h = jnp.dot(xn, w_ref[...], preferred_element_type=jnp.float32)
    hs = h * _dis_of(deg_ref[...])
    hs_ref[0] = hs[:, :HH]
    hs_ref[1] = hs[:, HH:]


def _mm1_call(x, W1, degcol):
    return pl.pallas_call(
        _mm1_body,
        grid=(NRB,),
        in_specs=[
            pl.BlockSpec((RB, D), lambda i: (i, 0)),
            pl.BlockSpec((D, H), lambda i: (0, 0)),
            pl.BlockSpec((RB, 1), lambda i: (i, 0)),
        ],
        out_specs=pl.BlockSpec((2, RB, HH), lambda i: (0, i, 0)),
        out_shape=jax.ShapeDtypeStruct((2, NP, HH), jnp.float32),
    )(x, W1, degcol)


def _bn_relu(z, acc, g_ref, be_ref):
    m = acc[0:1] * (1.0 / N)
    var = acc[1:2] * (1.0 / N) - m * m
    rs = lax.rsqrt(var + 1e-5)
    return jnp.maximum((z - m) * rs * g_ref[...] + be_ref[...], 0.0)


def _z_stats_phase(t_ref, hs_ref, deg_ref, b_ref, zbuf, acc, i):
    t = jnp.concatenate([t_ref[0], t_ref[1]], axis=1)
    hs = jnp.concatenate([hs_ref[0], hs_ref[1]], axis=1)
    z = _dis_of(deg_ref[...]) * (t + hs) + b_ref[...]
    zbuf[i] = z

    @pl.when(i == 0)
    def _():
        acc[...] = jnp.zeros_like(acc)

    acc[0:1] += jnp.sum(z, axis=0, keepdims=True)
    acc[1:2] += jnp.sum(z * z, axis=0, keepdims=True)


def _layer_body(t_ref, hs_ref, deg_ref, b_ref, g_ref, be_ref, w_ref,
                hs2_ref, zbuf, acc):
    p = pl.program_id(0)
    i = pl.program_id(1)

    @pl.when(p == 0)
    def _():
        _z_stats_phase(t_ref, hs_ref, deg_ref, b_ref, zbuf, acc, i)

    @pl.when(p == 1)
    def _():
        a = _bn_relu(zbuf[i], acc, g_ref, be_ref)
        h = jnp.dot(a, w_ref[...], preferred_element_type=jnp.float32)
        hs2 = h * _dis_of(deg_ref[...])
        hs2_ref[0] = hs2[:, :HH]
        hs2_ref[1] = hs2[:, HH:]


def _layer_call(tT, hsT, degcol, brow, grow, berow, W2):
    return pl.pallas_call(
        _layer_body,
        grid=(2, NRB),
        in_specs=[
            pl.BlockSpec((2, RB, HH), lambda p, i: (0, i * (1 - p), 0)),
            pl.BlockSpec((2, RB, HH), lambda p, i: (0, i * (1 - p), 0)),
            pl.BlockSpec((RB, 1), lambda p, i: (i, 0)),
            pl.BlockSpec((1, H), lambda p, i: (0, 0)),
            pl.BlockSpec((1, H), lambda p, i: (0, 0)),
            pl.BlockSpec((1, H), lambda p, i: (0, 0)),
            pl.BlockSpec((H, H), lambda p, i: (0, 0)),
        ],
        out_specs=pl.BlockSpec((2, RB, HH), lambda p, i: (0, i * p, 0)),
        out_shape=jax.ShapeDtypeStruct((2, NP, HH), jnp.float32),
        scratch_shapes=[
            pltpu.VMEM((NRB, RB, H), jnp.float32),
            pltpu.VMEM((2, H), jnp.float32),
        ],
    )(tT, hsT, degcol, brow, grow, berow, W2)


def _final_body(t_ref, hs_ref, deg_ref, b_ref, g_ref, be_ref, w_ref, bo_ref,
                o_ref, zbuf, acc):
    p = pl.program_id(0)
    i = pl.program_id(1)

    @pl.when(p == 0)
    def _():
        _z_stats_phase(t_ref, hs_ref, deg_ref, b_ref, zbuf, acc, i)

    @pl.when(p == 1)
    def _():
        a = _bn_relu(zbuf[i], acc, g_ref, be_ref)
        o = jnp.dot(a, w_ref[...], preferred_element_type=jnp.float32)
        o = o + bo_ref[...]
        m = jnp.max(o, axis=1, keepdims=True)
        e = jnp.exp(o - m)
        o_ref[...] = e / jnp.sum(e, axis=1, keepdims=True)


def _final_call(tT, hsT, degcol, brow, grow, berow, WoT, borow):
    return pl.pallas_call(
        _final_body,
        grid=(2, NRB),
        in_specs=[
            pl.BlockSpec((2, RB, HH), lambda p, i: (0, i * (1 - p), 0)),
            pl.BlockSpec((2, RB, HH), lambda p, i: (0, i * (1 - p), 0)),
            pl.BlockSpec((RB, 1), lambda p, i: (i, 0)),
            pl.BlockSpec((1, H), lambda p, i: (0, 0)),
            pl.BlockSpec((1, H), lambda p, i: (0, 0)),
            pl.BlockSpec((1, H), lambda p, i: (0, 0)),
            pl.BlockSpec((H, C), lambda p, i: (0, 0)),
            pl.BlockSpec((1, C), lambda p, i: (0, 0)),
        ],
        out_specs=pl.BlockSpec((RB, C), lambda p, i: (i * p, 0)),
        out_shape=jax.ShapeDtypeStruct((N, C), jnp.float32),
        scratch_shapes=[
            pltpu.VMEM((NRB, RB, H), jnp.float32),
            pltpu.VMEM((2, H), jnp.float32),
        ],
    )(tT, hsT, degcol, brow, grow, berow, WoT, borow)


def kernel(x, edge_index, W1, b1, g1, be1, W2, b2, g2, be2, Wo, bo):
    src = edge_index[0]
    dst = edge_index[1]

    ones_k = jnp.ones((K,), jnp.float32)
    zcol = jnp.zeros((RPT,), jnp.float32)
    zrows = jnp.zeros((RPT, HH), jnp.float32)

    dst_d = dst.reshape(NC * NS, NCHD, K)
    src_t = jnp.stack([src, src + NP]).reshape(NC * NS * NSEG, SEG, K)
    dst_t = dst.reshape(NS * NSEG, SEG, K)

    degp = _deg_call(dst_d, ones_k, zcol)
    degcol = (degp[:NP] + degp[NP:])[:N, None]

    hs1T = _mm1_call(x, W1, degcol)
    t1 = _agg_call(hs1T.reshape(NC * NP, HH), src_t, dst_t, zrows)
    hs2T = _layer_call(t1.reshape(NC, NP, HH), hs1T, degcol,
                       b1.reshape(1, H), g1.reshape(1, H),
                       be1.reshape(1, H), W2)
    t2 = _agg_call(hs2T.reshape(NC * NP, HH), src_t, dst_t, zrows)
    return _final_call(t2.reshape(NC, NP, HH), hs2T, degcol,
                       b2.reshape(1, H), g2.reshape(1, H),
                       be2.reshape(1, H), Wo.T, bo.reshape(1, C))


# speedup vs baseline: 19.4629x; 1.0015x over previous
import functools

import jax
import jax.numpy as jnp
from jax import lax
from jax.experimental import pallas as pl
from jax.experimental.pallas import tpu as pltpu
from jax.experimental.pallas import tpu_sc as plsc

N = 10000
E = 320000
D = 128
H = 256
C = 1000

NC = 2
NS = 16
NP = 10240
RPT = NP // NS
HH = H // 2

RB = 1000
NRB = N // RB

K = 80

@functools.cache
def _sc_mesh():
    return plsc.VectorSubcoreMesh(
        core_axis_name="c", subcore_axis_name="s", num_cores=NC, num_subcores=NS
    )


EW = E // (NC * NS)
NCHD = EW // K
DGRP = 5
ET = E // NS
NCH = ET // K
SEG = 10
NSEG = NCH // SEG


def _deg_body(dst_hbm, ones_hbm, zcol_hbm, out_hbm, idxb, ones_v, acc, sem):
    c = lax.axis_index("c")
    s = lax.axis_index("s")
    w = c * NS + s

    pltpu.sync_copy(zcol_hbm, acc.at[pl.ds(s * RPT, RPT)])
    pltpu.sync_copy(dst_hbm.at[w], idxb)
    pltpu.sync_copy(ones_hbm, ones_v)
    plsc.subcore_barrier()

    def body(g, carry):
        for b in range(DGRP):
            ch = g * DGRP + b
            pltpu.async_copy(ones_v, acc.at[idxb.at[ch]], sem, add=True)
        for b in range(DGRP):
            ch = g * DGRP + b
            pltpu.make_async_copy(ones_v, acc.at[idxb.at[ch]], sem).wait()
        return carry

    lax.fori_loop(0, NCHD // DGRP, body, 0)
    plsc.subcore_barrier()
    pltpu.sync_copy(
        acc.at[pl.ds(s * RPT, RPT)],
        out_hbm.at[pl.ds(c * NP + s * RPT, RPT)],
    )


def _deg_call(*args):
    return pl.kernel(
        _deg_body,
        out_type=jax.ShapeDtypeStruct((NC * NP,), jnp.float32),
        mesh=_sc_mesh(),
        scratch_types=[
            pltpu.VMEM((NCHD, K), jnp.int32),
            pltpu.VMEM((K,), jnp.float32),
            pltpu.VMEM_SHARED((NP,), jnp.float32),
            pltpu.SemaphoreType.DMA,
        ],
    )(*args)


def _agg_body(hs_hbm, src_hbm, dst_hbm, zrows_hbm, out_hbm,
              sidxb, didxb, rows, acc, gsem, ssem):
    c = lax.axis_index("c")
    s = lax.axis_index("s")

    pltpu.sync_copy(zrows_hbm, acc.at[pl.ds(s * RPT, RPT)])
    plsc.subcore_barrier()

    def seg_body(sg, carry):
        pltpu.sync_copy(src_hbm.at[(c * NS + s) * NSEG + sg], sidxb)
        pltpu.sync_copy(dst_hbm.at[s * NSEG + sg], didxb)

        pltpu.async_copy(hs_hbm.at[sidxb.at[0]], rows[0], gsem[0])
        pltpu.async_copy(hs_hbm.at[sidxb.at[1]], rows[1], gsem[1])
        for ch in range(SEG):
            b = ch % 3
            pltpu.make_async_copy(
                hs_hbm.at[sidxb.at[ch]], rows[b], gsem[b]
            ).wait()
            if ch >= 1:
                pb = (ch - 1) % 3
                pltpu.make_async_copy(
                    rows[pb], acc.at[didxb.at[ch - 1]], ssem[pb]
                ).wait()
            if ch + 2 < SEG:
                nb = (ch + 2) % 3
                pltpu.async_copy(hs_hbm.at[sidxb.at[ch + 2]], rows[nb],
                                 gsem[nb])
            pltpu.async_copy(rows[b], acc.at[didxb.at[ch]], ssem[b],
                             add=True)
        pltpu.make_async_copy(
            rows[(SEG - 1) % 3], acc.at[didxb.at[SEG - 1]],
            ssem[(SEG - 1) % 3]
        ).wait()
        return carry

    lax.fori_loop(0, NSEG, seg_body, 0)
    plsc.subcore_barrier()
    pltpu.sync_copy(
        acc.at[pl.ds(s * RPT, RPT)],
        out_hbm.at[pl.ds(c * NP + s * RPT, RPT)],
    )


def _agg_call(*args):
    return pl.kernel(
        _agg_body,
        out_type=jax.ShapeDtypeStruct((NC * NP, HH), jnp.float32),
        mesh=_sc_mesh(),
        scratch_types=[
            pltpu.VMEM((SEG, K), jnp.int32),
            pltpu.VMEM((SEG, K), jnp.int32),
            [pltpu.VMEM((K, HH), jnp.float32) for _ in range(3)],
            pltpu.VMEM_SHARED((NP, HH), jnp.float32),
            [pltpu.SemaphoreType.DMA for _ in range(3)],
            [pltpu.SemaphoreType.DMA for _ in range(3)],
        ],
    )(*args)


def _dis_of(degcol_blk):
    return lax.rsqrt(degcol_blk + 1.0)


def _mm1_body(x_ref, w_ref, deg_ref, hs_ref):
    x = x_ref[...]
    nrm = jnp.sqrt(jnp.sum(x * x, axis=1, keepdims=True))
    xn = x / jnp.maximum(nrm, 1e-12)
    h = jnp.dot(xn, w_ref[...], preferred_element_type=jnp.float32)
    hs = h * _dis_of(deg_ref[...])
    hs_ref[0] = hs[:, :HH]
    hs_ref[1] = hs[:, HH:]


def _mm1_call(x, W1, degcol):
    return pl.pallas_call(
        _mm1_body,
        grid=(NRB,),
        in_specs=[
            pl.BlockSpec((RB, D), lambda i: (i, 0)),
            pl.BlockSpec((D, H), lambda i: (0, 0)),
            pl.BlockSpec((RB, 1), lambda i: (i, 0)),
        ],
        out_specs=pl.BlockSpec((2, RB, HH), lambda i: (0, i, 0)),
        out_shape=jax.ShapeDtypeStruct((2, NP, HH), jnp.float32),
    )(x, W1, degcol)


def _bn_relu(z, acc, g_ref, be_ref):
    m = acc[0:1] * (1.0 / N)
    var = acc[1:2] * (1.0 / N) - m * m
    rs = lax.rsqrt(var + 1e-5)
    return jnp.maximum((z - m) * rs * g_ref[...] + be_ref[...], 0.0)


def _z_stats_phase(t_ref, hs_ref, deg_ref, b_ref, zbuf, acc, i):
    t = jnp.concatenate([t_ref[0], t_ref[1]], axis=1)
    hs = jnp.concatenate([hs_ref[0], hs_ref[1]], axis=1)
    z = _dis_of(deg_ref[...]) * (t + hs) + b_ref[...]
    zbuf[i] = z

    @pl.when(i == 0)
    def _():
        acc[...] = jnp.zeros_like(acc)

    acc[0:1] += jnp.sum(z, axis=0, keepdims=True)
    acc[1:2] += jnp.sum(z * z, axis=0, keepdims=True)


def _layer_body(t_ref, hs_ref, deg_ref, b_ref, g_ref, be_ref, w_ref,
                hs2_ref, zbuf, acc):
    p = pl.program_id(0)
    i = pl.program_id(1)

    @pl.when(p == 0)
    def _():
        _z_stats_phase(t_ref, hs_ref, deg_ref, b_ref, zbuf, acc, i)

    @pl.when(p == 1)
    def _():
        a = _bn_relu(zbuf[i], acc, g_ref, be_ref)
        h = jnp.dot(a, w_ref[...], preferred_element_type=jnp.float32)
        hs2 = h * _dis_of(deg_ref[...])
        hs2_ref[0] = hs2[:, :HH]
        hs2_ref[1] = hs2[:, HH:]


def _layer_call(tT, hsT, degcol, brow, grow, berow, W2):
    return pl.pallas_call(
        _layer_body,
        grid=(2, NRB),
        in_specs=[
            pl.BlockSpec((2, RB, HH), lambda p, i: (0, i * (1 - p), 0)),
            pl.BlockSpec((2, RB, HH), lambda p, i: (0, i * (1 - p), 0)),
            pl.BlockSpec((RB, 1), lambda p, i: (i, 0)),
            pl.BlockSpec((1, H), lambda p, i: (0, 0)),
            pl.BlockSpec((1, H), lambda p, i: (0, 0)),
            pl.BlockSpec((1, H), lambda p, i: (0, 0)),
            pl.BlockSpec((H, H), lambda p, i: (0, 0)),
        ],
        out_specs=pl.BlockSpec((2, RB, HH), lambda p, i: (0, i * p, 0)),
        out_shape=jax.ShapeDtypeStruct((2, NP, HH), jnp.float32),
        scratch_shapes=[
            pltpu.VMEM((NRB, RB, H), jnp.float32),
            pltpu.VMEM((2, H), jnp.float32),
        ],
    )(tT, hsT, degcol, brow, grow, berow, W2)


def _final_body(t_ref, hs_ref, deg_ref, b_ref, g_ref, be_ref, w_ref, bo_ref,
                o_ref, zbuf, acc):
    p = pl.program_id(0)
    i = pl.program_id(1)

    @pl.when(p == 0)
    def _():
        _z_stats_phase(t_ref, hs_ref, deg_ref, b_ref, zbuf, acc, i)

    @pl.when(p == 1)
    def _():
        a = _bn_relu(zbuf[i], acc, g_ref, be_ref)
        o = lax.dot_general(a, w_ref[...], (((1,), (1,)), ((), ())),
                            preferred_element_type=jnp.float32)
        o = o + bo_ref[...]
        m = jnp.max(o, axis=1, keepdims=True)
        e = jnp.exp(o - m)
        o_ref[...] = e / jnp.sum(e, axis=1, keepdims=True)


def _final_call(tT, hsT, degcol, brow, grow, berow, Wo, borow):
    return pl.pallas_call(
        _final_body,
        grid=(2, NRB),
        in_specs=[
            pl.BlockSpec((2, RB, HH), lambda p, i: (0, i * (1 - p), 0)),
            pl.BlockSpec((2, RB, HH), lambda p, i: (0, i * (1 - p), 0)),
            pl.BlockSpec((RB, 1), lambda p, i: (i, 0)),
            pl.BlockSpec((1, H), lambda p, i: (0, 0)),
            pl.BlockSpec((1, H), lambda p, i: (0, 0)),
            pl.BlockSpec((1, H), lambda p, i: (0, 0)),
            pl.BlockSpec((C, H), lambda p, i: (0, 0)),
            pl.BlockSpec((1, C), lambda p, i: (0, 0)),
        ],
        out_specs=pl.BlockSpec((RB, C), lambda p, i: (i * p, 0)),
        out_shape=jax.ShapeDtypeStruct((N, C), jnp.float32),
        scratch_shapes=[
            pltpu.VMEM((NRB, RB, H), jnp.float32),
            pltpu.VMEM((2, H), jnp.float32),
        ],
    )(tT, hsT, degcol, brow, grow, berow, Wo, borow)


def kernel(x, edge_index, W1, b1, g1, be1, W2, b2, g2, be2, Wo, bo):
    src = edge_index[0]
    dst = edge_index[1]

    ones_k = jnp.ones((K,), jnp.float32)
    zcol = jnp.zeros((RPT,), jnp.float32)
    zrows = jnp.zeros((RPT, HH), jnp.float32)

    dst_d = dst.reshape(NC * NS, NCHD, K)
    src_t = jnp.stack([src, src + NP]).reshape(NC * NS * NSEG, SEG, K)
    dst_t = dst.reshape(NS * NSEG, SEG, K)

    degp = _deg_call(dst_d, ones_k, zcol)
    degcol = (degp[:NP] + degp[NP:])[:N, None]

    hs1T = _mm1_call(x, W1, degcol)
    t1 = _agg_call(hs1T.reshape(NC * NP, HH), src_t, dst_t, zrows)
    hs2T = _layer_call(t1.reshape(NC, NP, HH), hs1T, degcol,
                       b1.reshape(1, H), g1.reshape(1, H),
                       be1.reshape(1, H), W2)
    t2 = _agg_call(hs2T.reshape(NC * NP, HH), src_t, dst_t, zrows)
    return _final_call(t2.reshape(NC, NP, HH), hs2T, degcol,
                       b2.reshape(1, H), g2.reshape(1, H),
                       be2.reshape(1, H), Wo, bo.reshape(1, C))


# speedup vs baseline: 21.4796x; 1.1036x over previous
import functools

import jax
import jax.numpy as jnp
from jax import lax
from jax.experimental import pallas as pl
from jax.experimental.pallas import tpu as pltpu
from jax.experimental.pallas import tpu_sc as plsc

N = 10000
E = 320000
D = 128
H = 256
C = 1000

NC = 2
NS = 16
NP = 10240
RPT = NP // NS
HH = H // 2

RB = 1000
NRB = N // RB

K = 80

@functools.cache
def _sc_mesh():
    return plsc.VectorSubcoreMesh(
        core_axis_name="c", subcore_axis_name="s", num_cores=NC, num_subcores=NS
    )


EW = E // (NC * NS)
NCHD = EW // K
DGRP = 5
ET = E // NS
NCH = ET // K
SEG = 25
NSEG = NCH // SEG


def _deg_body(dst_hbm, ones_hbm, zcol_hbm, out_hbm, idxb, ones_v, acc, sem):
    c = lax.axis_index("c")
    s = lax.axis_index("s")
    w = c * NS + s

    pltpu.sync_copy(zcol_hbm, acc.at[pl.ds(s * RPT, RPT)])
    pltpu.sync_copy(dst_hbm.at[w], idxb)
    pltpu.sync_copy(ones_hbm, ones_v)
    plsc.subcore_barrier()

    def body(g, carry):
        for b in range(DGRP):
            ch = g * DGRP + b
            pltpu.async_copy(ones_v, acc.at[idxb.at[ch]], sem, add=True)
        for b in range(DGRP):
            ch = g * DGRP + b
            pltpu.make_async_copy(ones_v, acc.at[idxb.at[ch]], sem).wait()
        return carry

    lax.fori_loop(0, NCHD // DGRP, body, 0)
    plsc.subcore_barrier()
    pltpu.sync_copy(
        acc.at[pl.ds(s * RPT, RPT)],
        out_hbm.at[pl.ds(c * NP + s * RPT, RPT)],
    )


def _deg_call(*args):
    return pl.kernel(
        _deg_body,
        out_type=jax.ShapeDtypeStruct((NC * NP,), jnp.float32),
        mesh=_sc_mesh(),
        scratch_types=[
            pltpu.VMEM((NCHD, K), jnp.int32),
            pltpu.VMEM((K,), jnp.float32),
            pltpu.VMEM_SHARED((NP,), jnp.float32),
            pltpu.SemaphoreType.DMA,
        ],
    )(*args)


def _agg_body(hs_hbm, src_hbm, dst_hbm, zrows_hbm, out_hbm,
              sidxb, didxb, rows, acc, gsem, ssem):
    c = lax.axis_index("c")
    s = lax.axis_index("s")

    pltpu.sync_copy(zrows_hbm, acc.at[pl.ds(s * RPT, RPT)])
    plsc.subcore_barrier()

    def seg_body(sg, carry):
        pltpu.sync_copy(src_hbm.at[(c * NS + s) * NSEG + sg], sidxb)
        pltpu.sync_copy(dst_hbm.at[s * NSEG + sg], didxb)

        pltpu.async_copy(hs_hbm.at[sidxb.at[0]], rows[0], gsem[0])
        pltpu.async_copy(hs_hbm.at[sidxb.at[1]], rows[1], gsem[1])
        for ch in range(SEG):
            b = ch % 3
            pltpu.make_async_copy(
                hs_hbm.at[sidxb.at[ch]], rows[b], gsem[b]
            ).wait()
            if ch >= 1:
                pb = (ch - 1) % 3
                pltpu.make_async_copy(
                    rows[pb], acc.at[didxb.at[ch - 1]], ssem[pb]
                ).wait()
            if ch + 2 < SEG:
                nb = (ch + 2) % 3
                pltpu.async_copy(hs_hbm.at[sidxb.at[ch + 2]], rows[nb],
                                 gsem[nb])
            pltpu.async_copy(rows[b], acc.at[didxb.at[ch]], ssem[b],
                             add=True)
        pltpu.make_async_copy(
            rows[(SEG - 1) % 3], acc.at[didxb.at[SEG - 1]],
            ssem[(SEG - 1) % 3]
        ).wait()
        return carry

    lax.fori_loop(0, NSEG, seg_body, 0)
    plsc.subcore_barrier()
    pltpu.sync_copy(
        acc.at[pl.ds(s * RPT, RPT)],
        out_hbm.at[pl.ds(c * NP + s * RPT, RPT)],
    )


def _agg_call(*args):
    return pl.kernel(
        _agg_body,
        out_type=jax.ShapeDtypeStruct((NC * NP, HH), jnp.float32),
        mesh=_sc_mesh(),
        scratch_types=[
            pltpu.VMEM((SEG, K), jnp.int32),
            pltpu.VMEM((SEG, K), jnp.int32),
            [pltpu.VMEM((K, HH), jnp.float32) for _ in range(3)],
            pltpu.VMEM_SHARED((NP, HH), jnp.float32),
            [pltpu.SemaphoreType.DMA for _ in range(3)],
            [pltpu.SemaphoreType.DMA for _ in range(3)],
        ],
    )(*args)


def _dis_of(degcol_blk):
    return lax.rsqrt(degcol_blk + 1.0)


def _mm1_body(x_ref, w_ref, deg_ref, hs_ref):
    x = x_ref[...]
    nrm = jnp.sqrt(jnp.sum(x * x, axis=1, keepdims=True))
    xn = x / jnp.maximum(nrm, 1e-12)
    h = jnp.dot(xn, w_ref[...], preferred_element_type=jnp.float32)
    hs = h * _dis_of(deg_ref[...])
    hs_ref[0] = hs[:, :HH]
    hs_ref[1] = hs[:, HH:]


def _mm1_call(x, W1, degcol):
    return pl.pallas_call(
        _mm1_body,
        grid=(NRB,),
        in_specs=[
            pl.BlockSpec((RB, D), lambda i: (i, 0)),
            pl.BlockSpec((D, H), lambda i: (0, 0)),
            pl.BlockSpec((RB, 1), lambda i: (i, 0)),
        ],
        out_specs=pl.BlockSpec((2, RB, HH), lambda i: (0, i, 0)),
        out_shape=jax.ShapeDtypeStruct((2, NP, HH), jnp.float32),
    )(x, W1, degcol)


def _bn_relu(z, acc, g_ref, be_ref):
    m = acc[0:1] * (1.0 / N)
    var = acc[1:2] * (1.0 / N) - m * m
    rs = lax.rsqrt(var + 1e-5)
    return jnp.maximum((z - m) * rs * g_ref[...] + be_ref[...], 0.0)


def _z_stats_phase(t_ref, hs_ref, deg_ref, b_ref, zbuf, acc, i):
    t = jnp.concatenate([t_ref[0], t_ref[1]], axis=1)
    hs = jnp.concatenate([hs_ref[0], hs_ref[1]], axis=1)
    z = _dis_of(deg_ref[...]) * (t + hs) + b_ref[...]
    zbuf[i] = z

    @pl.when(i == 0)
    def _():
        acc[...] = jnp.zeros_like(acc)

    acc[0:1] += jnp.sum(z, axis=0, keepdims=True)
    acc[1:2] += jnp.sum(z * z, axis=0, keepdims=True)


def _layer_body(t_ref, hs_ref, deg_ref, b_ref, g_ref, be_ref, w_ref,
                hs2_ref, zbuf, acc):
    p = pl.program_id(0)
    i = pl.program_id(1)

    @pl.when(p == 0)
    def _():
        _z_stats_phase(t_ref, hs_ref, deg_ref, b_ref, zbuf, acc, i)

    @pl.when(p == 1)
    def _():
        a = _bn_relu(zbuf[i], acc, g_ref, be_ref)
        h = jnp.dot(a, w_ref[...], preferred_element_type=jnp.float32)
        hs2 = h * _dis_of(deg_ref[...])
        hs2_ref[0] = hs2[:, :HH]
        hs2_ref[1] = hs2[:, HH:]


def _layer_call(tT, hsT, degcol, brow, grow, berow, W2):
    return pl.pallas_call(
        _layer_body,
        grid=(2, NRB),
        in_specs=[
            pl.BlockSpec((2, RB, HH), lambda p, i: (0, i * (1 - p), 0)),
            pl.BlockSpec((2, RB, HH), lambda p, i: (0, i * (1 - p), 0)),
            pl.BlockSpec((RB, 1), lambda p, i: (i, 0)),
            pl.BlockSpec((1, H), lambda p, i: (0, 0)),
            pl.BlockSpec((1, H), lambda p, i: (0, 0)),
            pl.BlockSpec((1, H), lambda p, i: (0, 0)),
            pl.BlockSpec((H, H), lambda p, i: (0, 0)),
        ],
        out_specs=pl.BlockSpec((2, RB, HH), lambda p, i: (0, i * p, 0)),
        out_shape=jax.ShapeDtypeStruct((2, NP, HH), jnp.float32),
        scratch_shapes=[
            pltpu.VMEM((NRB, RB, H), jnp.float32),
            pltpu.VMEM((2, H), jnp.float32),
        ],
    )(tT, hsT, degcol, brow, grow, berow, W2)


def _final_body(t_ref, hs_ref, deg_ref, b_ref, g_ref, be_ref, w_ref, bo_ref,
                o_ref, zbuf, acc):
    p = pl.program_id(0)
    i = pl.program_id(1)

    @pl.when(p == 0)
    def _():
        _z_stats_phase(t_ref, hs_ref, deg_ref, b_ref, zbuf, acc, i)

    @pl.when(p == 1)
    def _():
        a = _bn_relu(zbuf[i], acc, g_ref, be_ref)
        o = lax.dot_general(a, w_ref[...], (((1,), (1,)), ((), ())),
                            preferred_element_type=jnp.float32)
        o = o + bo_ref[...]
        m = jnp.max(o, axis=1, keepdims=True)
        e = jnp.exp(o - m)
        o_ref[...] = e / jnp.sum(e, axis=1, keepdims=True)


def _final_call(tT, hsT, degcol, brow, grow, berow, Wo, borow):
    return pl.pallas_call(
        _final_body,
        grid=(2, NRB),
        in_specs=[
            pl.BlockSpec((2, RB, HH), lambda p, i: (0, i * (1 - p), 0)),
            pl.BlockSpec((2, RB, HH), lambda p, i: (0, i * (1 - p), 0)),
            pl.BlockSpec((RB, 1), lambda p, i: (i, 0)),
            pl.BlockSpec((1, H), lambda p, i: (0, 0)),
            pl.BlockSpec((1, H), lambda p, i: (0, 0)),
            pl.BlockSpec((1, H), lambda p, i: (0, 0)),
            pl.BlockSpec((C, H), lambda p, i: (0, 0)),
            pl.BlockSpec((1, C), lambda p, i: (0, 0)),
        ],
        out_specs=pl.BlockSpec((RB, C), lambda p, i: (i * p, 0)),
        out_shape=jax.ShapeDtypeStruct((N, C), jnp.float32),
        scratch_shapes=[
            pltpu.VMEM((NRB, RB, H), jnp.float32),
            pltpu.VMEM((2, H), jnp.float32),
        ],
    )(tT, hsT, degcol, brow, grow, berow, Wo, borow)


def kernel(x, edge_index, W1, b1, g1, be1, W2, b2, g2, be2, Wo, bo):
    src = edge_index[0]
    dst = edge_index[1]

    ones_k = jnp.ones((K,), jnp.float32)
    zcol = jnp.zeros((RPT,), jnp.float32)
    zrows = jnp.zeros((RPT, HH), jnp.float32)

    dst_d = dst.reshape(NC * NS, NCHD, K)
    src_t = jnp.stack([src, src + NP]).reshape(NC * NS * NSEG, SEG, K)
    dst_t = dst.reshape(NS * NSEG, SEG, K)

    degp = _deg_call(dst_d, ones_k, zcol)
    degcol = (degp[:NP] + degp[NP:])[:N, None]

    hs1T = _mm1_call(x, W1, degcol)
    t1 = _agg_call(hs1T.reshape(NC * NP, HH), src_t, dst_t, zrows)
    hs2T = _layer_call(t1.reshape(NC, NP, HH), hs1T, degcol,
                       b1.reshape(1, H), g1.reshape(1, H),
                       be1.reshape(1, H), W2)
    t2 = _agg_call(hs2T.reshape(NC * NP, HH), src_t, dst_t, zrows)
    return _final_call(t2.reshape(NC, NP, HH), hs2T, degcol,
                       b2.reshape(1, H), g2.reshape(1, H),
                       be2.reshape(1, H), Wo, bo.reshape(1, C))


# speedup vs baseline: 22.2578x; 1.0362x over previous
import functools

import jax
import jax.numpy as jnp
from jax import lax
from jax.experimental import pallas as pl
from jax.experimental.pallas import tpu as pltpu
from jax.experimental.pallas import tpu_sc as plsc

N = 10000
E = 320000
D = 128
H = 256
C = 1000

NC = 2
NS = 16
NP = 10240
RPT = NP // NS
HH = H // 2

RB = 1000
NRB = N // RB

K = 80

@functools.cache
def _sc_mesh():
    return plsc.VectorSubcoreMesh(
        core_axis_name="c", subcore_axis_name="s", num_cores=NC, num_subcores=NS
    )


EW = E // (NC * NS)
NCHD = EW // K
DGRP = 5
ET = E // NS
NCH = ET // K
SEG = 50
NSEG = NCH // SEG


def _deg_body(dst_hbm, ones_hbm, zcol_hbm, out_hbm, idxb, ones_v, acc, sem):
    c = lax.axis_index("c")
    s = lax.axis_index("s")
    w = c * NS + s

    pltpu.sync_copy(zcol_hbm, acc.at[pl.ds(s * RPT, RPT)])
    pltpu.sync_copy(dst_hbm.at[w], idxb)
    pltpu.sync_copy(ones_hbm, ones_v)
    plsc.subcore_barrier()

    def body(g, carry):
        for b in range(DGRP):
            ch = g * DGRP + b
            pltpu.async_copy(ones_v, acc.at[idxb.at[ch]], sem, add=True)
        for b in range(DGRP):
            ch = g * DGRP + b
            pltpu.make_async_copy(ones_v, acc.at[idxb.at[ch]], sem).wait()
        return carry

    lax.fori_loop(0, NCHD // DGRP, body, 0)
    plsc.subcore_barrier()
    pltpu.sync_copy(
        acc.at[pl.ds(s * RPT, RPT)],
        out_hbm.at[pl.ds(c * NP + s * RPT, RPT)],
    )


def _deg_call(*args):
    return pl.kernel(
        _deg_body,
        out_type=jax.ShapeDtypeStruct((NC * NP,), jnp.float32),
        mesh=_sc_mesh(),
        scratch_types=[
            pltpu.VMEM((NCHD, K), jnp.int32),
            pltpu.VMEM((K,), jnp.float32),
            pltpu.VMEM_SHARED((NP,), jnp.float32),
            pltpu.SemaphoreType.DMA,
        ],
    )(*args)


def _agg_body(hs_hbm, src_hbm, dst_hbm, zrows_hbm, out_hbm,
              sidxb, didxb, rows, acc, gsem, ssem):
    c = lax.axis_index("c")
    s = lax.axis_index("s")

    pltpu.sync_copy(zrows_hbm, acc.at[pl.ds(s * RPT, RPT)])
    plsc.subcore_barrier()

    def seg_body(sg, carry):
        pltpu.sync_copy(src_hbm.at[(c * NS + s) * NSEG + sg], sidxb)
        pltpu.sync_copy(dst_hbm.at[s * NSEG + sg], didxb)

        pltpu.async_copy(hs_hbm.at[sidxb.at[0]], rows[0], gsem[0])
        pltpu.async_copy(hs_hbm.at[sidxb.at[1]], rows[1], gsem[1])
        for ch in range(SEG):
            b = ch % 3
            pltpu.make_async_copy(
                hs_hbm.at[sidxb.at[ch]], rows[b], gsem[b]
            ).wait()
            if ch >= 1:
                pb = (ch - 1) % 3
                pltpu.make_async_copy(
                    rows[pb], acc.at[didxb.at[ch - 1]], ssem[pb]
                ).wait()
            if ch + 2 < SEG:
                nb = (ch + 2) % 3
                pltpu.async_copy(hs_hbm.at[sidxb.at[ch + 2]], rows[nb],
                                 gsem[nb])
            pltpu.async_copy(rows[b], acc.at[didxb.at[ch]], ssem[b],
                             add=True)
        pltpu.make_async_copy(
            rows[(SEG - 1) % 3], acc.at[didxb.at[SEG - 1]],
            ssem[(SEG - 1) % 3]
        ).wait()
        return carry

    lax.fori_loop(0, NSEG, seg_body, 0)
    plsc.subcore_barrier()
    pltpu.sync_copy(
        acc.at[pl.ds(s * RPT, RPT)],
        out_hbm.at[pl.ds(c * NP + s * RPT, RPT)],
    )


def _agg_call(*args):
    return pl.kernel(
        _agg_body,
        out_type=jax.ShapeDtypeStruct((NC * NP, HH), jnp.float32),
        mesh=_sc_mesh(),
        scratch_types=[
            pltpu.VMEM((SEG, K), jnp.int32),
            pltpu.VMEM((SEG, K), jnp.int32),
            [pltpu.VMEM((K, HH), jnp.float32) for _ in range(3)],
            pltpu.VMEM_SHARED((NP, HH), jnp.float32),
            [pltpu.SemaphoreType.DMA for _ in range(3)],
            [pltpu.SemaphoreType.DMA for _ in range(3)],
        ],
    )(*args)


def _dis_of(degcol_blk):
    return lax.rsqrt(degcol_blk + 1.0)


def _mm1_body(x_ref, w_ref, deg_ref, hs_ref):
    x = x_ref[...]
    nrm = jnp.sqrt(jnp.sum(x * x, axis=1, keepdims=True))
    xn = x / jnp.maximum(nrm, 1e-12)
    h = jnp.dot(xn, w_ref[...], preferred_element_type=jnp.float32)
    hs = h * _dis_of(deg_ref[...])
    hs_ref[0] = hs[:, :HH]
    hs_ref[1] = hs[:, HH:]


def _mm1_call(x, W1, degcol):
    return pl.pallas_call(
        _mm1_body,
        grid=(NRB,),
        in_specs=[
            pl.BlockSpec((RB, D), lambda i: (i, 0)),
            pl.BlockSpec((D, H), lambda i: (0, 0)),
            pl.BlockSpec((RB, 1), lambda i: (i, 0)),
        ],
        out_specs=pl.BlockSpec((2, RB, HH), lambda i: (0, i, 0)),
        out_shape=jax.ShapeDtypeStruct((2, NP, HH), jnp.float32),
    )(x, W1, degcol)


def _bn_relu(z, acc, g_ref, be_ref):
    m = acc[0:1] * (1.0 / N)
    var = acc[1:2] * (1.0 / N) - m * m
    rs = lax.rsqrt(var + 1e-5)
    return jnp.maximum((z - m) * rs * g_ref[...] + be_ref[...], 0.0)


def _z_stats_phase(t_ref, hs_ref, deg_ref, b_ref, zbuf, acc, i):
    t = jnp.concatenate([t_ref[0], t_ref[1]], axis=1)
    hs = jnp.concatenate([hs_ref[0], hs_ref[1]], axis=1)
    z = _dis_of(deg_ref[...]) * (t + hs) + b_ref[...]
    zbuf[i] = z

    @pl.when(i == 0)
    def _():
        acc[...] = jnp.zeros_like(acc)

    acc[0:1] += jnp.sum(z, axis=0, keepdims=True)
    acc[1:2] += jnp.sum(z * z, axis=0, keepdims=True)


def _layer_body(t_ref, hs_ref, deg_ref, b_ref, g_ref, be_ref, w_ref,
                hs2_ref, zbuf, acc):
    p = pl.program_id(0)
    i = pl.program_id(1)

    @pl.when(p == 0)
    def _():
        _z_stats_phase(t_ref, hs_ref, deg_ref, b_ref, zbuf, acc, i)

    @pl.when(p == 1)
    def _():
        a = _bn_relu(zbuf[i], acc, g_ref, be_ref)
        h = jnp.dot(a, w_ref[...], preferred_element_type=jnp.float32)
        hs2 = h * _dis_of(deg_ref[...])
        hs2_ref[0] = hs2[:, :HH]
        hs2_ref[1] = hs2[:, HH:]


def _layer_call(tT, hsT, degcol, brow, grow, berow, W2):
    return pl.pallas_call(
        _layer_body,
        grid=(2, NRB),
        in_specs=[
            pl.BlockSpec((2, RB, HH), lambda p, i: (0, i * (1 - p), 0)),
            pl.BlockSpec((2, RB, HH), lambda p, i: (0, i * (1 - p), 0)),
            pl.BlockSpec((RB, 1), lambda p, i: (i, 0)),
            pl.BlockSpec((1, H), lambda p, i: (0, 0)),
            pl.BlockSpec((1, H), lambda p, i: (0, 0)),
            pl.BlockSpec((1, H), lambda p, i: (0, 0)),
            pl.BlockSpec((H, H), lambda p, i: (0, 0)),
        ],
        out_specs=pl.BlockSpec((2, RB, HH), lambda p, i: (0, i * p, 0)),
        out_shape=jax.ShapeDtypeStruct((2, NP, HH), jnp.float32),
        scratch_shapes=[
            pltpu.VMEM((NRB, RB, H), jnp.float32),
            pltpu.VMEM((2, H), jnp.float32),
        ],
    )(tT, hsT, degcol, brow, grow, berow, W2)


def _final_body(t_ref, hs_ref, deg_ref, b_ref, g_ref, be_ref, w_ref, bo_ref,
                o_ref, zbuf, acc):
    p = pl.program_id(0)
    i = pl.program_id(1)

    @pl.when(p == 0)
    def _():
        _z_stats_phase(t_ref, hs_ref, deg_ref, b_ref, zbuf, acc, i)

    @pl.when(p == 1)
    def _():
        a = _bn_relu(zbuf[i], acc, g_ref, be_ref)
        o = lax.dot_general(a, w_ref[...], (((1,), (1,)), ((), ())),
                            preferred_element_type=jnp.float32)
        o = o + bo_ref[...]
        m = jnp.max(o, axis=1, keepdims=True)
        e = jnp.exp(o - m)
        o_ref[...] = e / jnp.sum(e, axis=1, keepdims=True)


def _final_call(tT, hsT, degcol, brow, grow, berow, Wo, borow):
    return pl.pallas_call(
        _final_body,
        grid=(2, NRB),
        in_specs=[
            pl.BlockSpec((2, RB, HH), lambda p, i: (0, i * (1 - p), 0)),
            pl.BlockSpec((2, RB, HH), lambda p, i: (0, i * (1 - p), 0)),
            pl.BlockSpec((RB, 1), lambda p, i: (i, 0)),
            pl.BlockSpec((1, H), lambda p, i: (0, 0)),
            pl.BlockSpec((1, H), lambda p, i: (0, 0)),
            pl.BlockSpec((1, H), lambda p, i: (0, 0)),
            pl.BlockSpec((C, H), lambda p, i: (0, 0)),
            pl.BlockSpec((1, C), lambda p, i: (0, 0)),
        ],
        out_specs=pl.BlockSpec((RB, C), lambda p, i: (i * p, 0)),
        out_shape=jax.ShapeDtypeStruct((N, C), jnp.float32),
        scratch_shapes=[
            pltpu.VMEM((NRB, RB, H), jnp.float32),
            pltpu.VMEM((2, H), jnp.float32),
        ],
    )(tT, hsT, degcol, brow, grow, berow, Wo, borow)


def kernel(x, edge_index, W1, b1, g1, be1, W2, b2, g2, be2, Wo, bo):
    src = edge_index[0]
    dst = edge_index[1]

    ones_k = jnp.ones((K,), jnp.float32)
    zcol = jnp.zeros((RPT,), jnp.float32)
    zrows = jnp.zeros((RPT, HH), jnp.float32)

    dst_d = dst.reshape(NC * NS, NCHD, K)
    src_t = jnp.stack([src, src + NP]).reshape(NC * NS * NSEG, SEG, K)
    dst_t = dst.reshape(NS * NSEG, SEG, K)

    degp = _deg_call(dst_d, ones_k, zcol)
    degcol = (degp[:NP] + degp[NP:])[:N, None]

    hs1T = _mm1_call(x, W1, degcol)
    t1 = _agg_call(hs1T.reshape(NC * NP, HH), src_t, dst_t, zrows)
    hs2T = _layer_call(t1.reshape(NC, NP, HH), hs1T, degcol,
                       b1.reshape(1, H), g1.reshape(1, H),
                       be1.reshape(1, H), W2)
    t2 = _agg_call(hs2T.reshape(NC * NP, HH), src_t, dst_t, zrows)
    return _final_call(t2.reshape(NC, NP, HH), hs2T, degcol,
                       b2.reshape(1, H), g2.reshape(1, H),
                       be2.reshape(1, H), Wo, bo.reshape(1, C))
